# baseline (speedup 1.0000x reference)
import functools

import jax
import jax.numpy as jnp
from jax import lax
from jax.experimental import pallas as pl
from jax.experimental.pallas import tpu as pltpu
from jax.experimental.pallas import tpu_sc as plsc

N_NODES = 10000
N_EDGES = 320000
DIM = 128
LANES = 16

CHUNK = 128
NC, NS = 2, 16
NW = NC * NS
C0 = 103
C1 = 54
TOTCH = NS * (C0 + C1)
PAD_TOTAL = TOTCH * CHUNK - N_EDGES

N_PAD = N_NODES + 8
NROWCH = -(-N_PAD // CHUNK)
Z_TAIL = N_PAD - (NROWCH - 1) * CHUNK
W_TAIL = N_NODES - (NROWCH - 1) * CHUNK
WCH = -(-NROWCH // NS)
H_PAD = N_NODES + LANES

RB = 400

_MESH = dict(core_axis_name="c", subcore_axis_name="s")


def _zero_rows(buf, ncols):
    zero16 = jnp.zeros((LANES,), jnp.float32)

    def zrow(i, _):
        row = buf.at[i]

        def zcol(j, _):
            row[pl.ds(j * LANES, LANES)] = zero16
            return 0
        lax.fori_loop(0, ncols // LANES, zcol, 0)
        return 0
    lax.fori_loop(0, CHUNK, zrow, 0)


def _init_shared(sh, buf, s, tail):
    def zsh(j, _):
        jj = j * NS + s

        @pl.when(jj < NROWCH - 1)
        def _():
            pltpu.sync_copy(buf, sh.at[pl.ds(jj * CHUNK, CHUNK)])

        @pl.when(jj == NROWCH - 1)
        def _():
            pltpu.sync_copy(buf.at[pl.ds(0, tail)],
                            sh.at[pl.ds(jj * CHUNK, tail)])
        return 0
    lax.fori_loop(0, WCH, zsh, 0)


def _write_shared(sh, out, c, s):
    def wout(j, _):
        jj = j * NS + s

        @pl.when(jj < NROWCH - 1)
        def _():
            pltpu.sync_copy(sh.at[pl.ds(jj * CHUNK, CHUNK)],
                            out.at[c, pl.ds(jj * CHUNK, CHUNK)])

        @pl.when(jj == NROWCH - 1)
        def _():
            pltpu.sync_copy(sh.at[pl.ds(jj * CHUNK, W_TAIL)],
                            out.at[c, pl.ds(jj * CHUNK, W_TAIL)])
        return 0
    lax.fori_loop(0, WCH, wout, 0)


def _sc_agg_body(with_cnt, h_hbm, pair_hbm, *refs):
    if with_cnt:
        agg_out, cnt_out, pairs, rows, agg_sh, hist, gsem, isem, ssem = refs
    else:
        agg_out, pairs, rows, agg_sh, gsem, isem, ssem = refs
        cnt_out = hist = None

    c = lax.axis_index("c")
    s = lax.axis_index("s")

    _zero_rows(rows.at[0], DIM)
    _init_shared(agg_sh, rows.at[0], s, Z_TAIL)

    if with_cnt:
        zero16 = jnp.zeros((LANES,), jnp.float32)

        def zhist(i, _):
            hist[pl.ds(i * LANES, LANES)] = zero16
            return 0
        lax.fori_loop(0, H_PAD // LANES, zhist, 0)

    ones16 = jnp.ones((LANES,), jnp.float32)

    plsc.subcore_barrier()

    base = jnp.where(c == 0, s * C0, NS * C0 + s * C1)
    nmy = jnp.where(c == 0, C0, C1)

    pltpu.sync_copy(pair_hbm.at[base], pairs.at[0])
    pltpu.async_copy(pair_hbm.at[base + 1], pairs.at[1], isem)
    pltpu.async_copy(h_hbm.at[pairs.at[0, 0]], rows.at[0], gsem)

    def step(t, _):
        par = lax.rem(t, 2)
        nxt = lax.rem(t + 1, 2)
        cur3 = lax.rem(t, 3)
        nxt3 = lax.rem(t + 1, 3)
        fut3 = lax.rem(t + 2, 3)

        @pl.when(t < nmy)
        def _():
            @pl.when(t + 1 < nmy)
            def _():
                pltpu.make_async_copy(pair_hbm.at[base + t + 1],
                                      pairs.at[nxt3], isem).wait()

            pltpu.make_async_copy(h_hbm.at[pairs.at[cur3, 0]],
                                  rows.at[par], gsem).wait()

            @pl.when(t > 0)
            def _():
                pltpu.make_async_copy(rows.at[nxt],
                                      agg_sh.at[pairs.at[fut3, 1]],
                                      ssem).wait()

            @pl.when(t + 1 < nmy)
            def _():
                pltpu.async_copy(h_hbm.at[pairs.at[nxt3, 0]], rows.at[nxt],
                                 gsem)

            pltpu.async_copy(rows.at[par], agg_sh.at[pairs.at[cur3, 1]],
                             ssem, add=True)

            @pl.when(t + 2 < nmy)
            def _():
                pltpu.async_copy(pair_hbm.at[base + t + 2], pairs.at[fut3],
                                 isem)

            if with_cnt:
                drow = pairs.at[cur3, 1]
                for k in range(CHUNK // LANES):
                    iv = drow[pl.ds(k * LANES, LANES)]
                    plsc.addupdate_scatter(hist, [iv], ones16)
        return 0
    lax.fori_loop(0, C0, step, 0)

    lpar = lax.rem(nmy - 1, 2)
    lcur3 = lax.rem(nmy - 1, 3)
    pltpu.make_async_copy(rows.at[lpar], agg_sh.at[pairs.at[lcur3, 1]],
                          ssem).wait()

    if with_cnt:
        wid = s * NC + c
        pltpu.sync_copy(hist.at[pl.ds(0, N_NODES)], cnt_out.at[wid, 0])

    plsc.subcore_barrier()
    _write_shared(agg_sh, agg_out, c, s)


def _make_sc_agg(with_cnt):
    if with_cnt:
        out_type = (jax.ShapeDtypeStruct((NC, N_NODES, DIM), jnp.float32),
                    jax.ShapeDtypeStruct((NW, 1, N_NODES), jnp.float32))
    else:
        out_type = jax.ShapeDtypeStruct((NC, N_NODES, DIM), jnp.float32)
    scratch = [
        pltpu.VMEM((3, 2, CHUNK), jnp.int32),
        pltpu.VMEM((2, CHUNK, DIM), jnp.float32),
        pltpu.VMEM_SHARED((N_PAD, DIM), jnp.float32),
    ]
    if with_cnt:
        scratch.append(pltpu.VMEM((H_PAD,), jnp.float32))
    scratch += [
        pltpu.SemaphoreType.DMA,
        pltpu.SemaphoreType.DMA,
        pltpu.SemaphoreType.DMA,
    ]
    params = (pltpu.CompilerParams(needs_layout_passes=False)
              if with_cnt else None)
    return pl.kernel(
        functools.partial(_sc_agg_body, with_cnt),
        out_type=out_type,
        mesh=plsc.VectorSubcoreMesh(**_MESH),
        compiler_params=params,
        scratch_types=scratch)


def _tc_layer_body(relu, p_ref, c_ref, x_ref, wl_ref, bl_ref, wr_ref, o_ref):
    deg = jnp.maximum(jnp.sum(c_ref[...], axis=1), 1.0)
    mean = (p_ref[0] + p_ref[1]) / deg[:, None]
    acc = jnp.dot(mean, wl_ref[...], preferred_element_type=jnp.float32)
    acc = acc + bl_ref[...]
    acc = acc + jnp.dot(x_ref[...], wr_ref[...],
                        preferred_element_type=jnp.float32)
    o_ref[...] = jnp.maximum(acc, 0.0) if relu else acc


def _tc_layer(p, cnt, x, Wl, bl, Wr, relu):
    return pl.pallas_call(
        functools.partial(_tc_layer_body, relu),
        grid=(N_NODES // RB,),
        in_specs=[
            pl.BlockSpec((NC, RB, DIM), lambda i: (0, i, 0)),
            pl.BlockSpec((RB, NW), lambda i: (i, 0)),
            pl.BlockSpec((RB, DIM), lambda i: (i, 0)),
            pl.BlockSpec((DIM, DIM), lambda i: (0, 0)),
            pl.BlockSpec((1, DIM), lambda i: (0, 0)),
            pl.BlockSpec((DIM, DIM), lambda i: (0, 0)),
        ],
        out_specs=pl.BlockSpec((RB, DIM), lambda i: (i, 0)),
        out_shape=jax.ShapeDtypeStruct((N_NODES, DIM), jnp.float32),
    )(p, cnt, x, Wl, bl.reshape(1, DIM), Wr)


def kernel(x, edge_index, edge_weight, Wl1, bl1, Wr1, Wl2, bl2, Wr2):
    del edge_weight
    src = jnp.concatenate(
        [edge_index[0], jnp.zeros((PAD_TOTAL,), jnp.int32)]
    ).reshape(TOTCH, 1, CHUNK)
    dst = jnp.concatenate(
        [edge_index[1], jnp.full((PAD_TOTAL,), N_NODES, jnp.int32)]
    ).reshape(TOTCH, 1, CHUNK)
    pair = jnp.concatenate([src, dst], axis=1)
    agg1, cnt3 = _make_sc_agg(True)(x, pair)
    cnt = cnt3.reshape(NW, N_NODES).T
    h = _tc_layer(agg1, cnt, x, Wl1, bl1, Wr1, relu=True)
    agg2 = _make_sc_agg(False)(h, pair)
    out = _tc_layer(agg2, cnt, h, Wl2, bl2, Wr2, relu=False)
    return out
